# baseline (speedup 1.0000x reference)
import functools

import jax
import jax.numpy as jnp
from jax import lax
from jax.experimental import pallas as pl
from jax.experimental.pallas import tpu as pltpu
from jax.experimental.pallas import tpu_sc as plsc

EMBED_DIM = 64
NUM_CONCEPTS = 36
NUM_CORES = 2
NUM_SUBCORES = 16
NUM_WORKERS = NUM_CORES * NUM_SUBCORES
GROUP = 8
NBUF = 2


def _lookup(table, idx3):
    nw, rows_w, ncol = idx3.shape
    nrows = nw * rows_w
    ngroups = rows_w // GROUP
    mesh = plsc.VectorSubcoreMesh(core_axis_name="c", subcore_axis_name="s")

    @functools.partial(
        pl.kernel,
        out_type=jax.ShapeDtypeStruct((nrows, ncol, EMBED_DIM), jnp.float32),
        mesh=mesh,
        scratch_types=[
            pltpu.VMEM((rows_w, ncol), jnp.int32),
            pltpu.VMEM((GROUP, ncol, EMBED_DIM), jnp.float32),
            pltpu.VMEM((GROUP, ncol, EMBED_DIM), jnp.float32),
            pltpu.VMEM_SHARED((NUM_CONCEPTS, EMBED_DIM), jnp.float32),
            pltpu.SemaphoreType.DMA,
            pltpu.SemaphoreType.DMA,
            pltpu.SemaphoreType.DMA,
            pltpu.SemaphoreType.DMA,
        ],
        compiler_params=pltpu.CompilerParams(use_tc_tiling_on_sc=False),
    )
    def k(table_hbm, idx_hbm, out_hbm, idx_v, buf0, buf1, table_sh,
          gs0, gs1, ss0, ss1):
        sid = lax.axis_index("s")
        wid = sid * NUM_CORES + lax.axis_index("c")

        @pl.when(sid == 0)
        def _():
            pltpu.sync_copy(table_hbm, table_sh)

        pltpu.sync_copy(idx_hbm.at[wid], idx_v)
        plsc.subcore_barrier()
        base_row = wid * rows_w
        bufs = (buf0, buf1)
        gsems = (gs0, gs1)
        ssems = (ss0, ss1)

        def fire_group(p, bi):
            for s in range(GROUP):
                pltpu.async_copy(
                    table_sh.at[idx_v.at[p * GROUP + s]],
                    bufs[bi].at[s],
                    gsems[bi],
                )

        for bi in range(NBUF):
            fire_group(bi, bi)

        def outer(g, carry):
            for bi in range(NBUF):
                p = g + bi
                pltpu.make_async_copy(
                    out_hbm.at[pl.ds(0, GROUP)], bufs[bi], gsems[bi]
                ).wait()
                out_slice = out_hbm.at[pl.ds(base_row + p * GROUP, GROUP)]
                sc = pltpu.async_copy(bufs[bi], out_slice, ssems[bi])
                sc.wait()

                @pl.when(p + NBUF < ngroups)
                def _():
                    fire_group(p + NBUF, bi)

            return carry

        lax.fori_loop(0, ngroups // NBUF, lambda i, c: outer(i * NBUF, c), 0)

    return k(table, idx3)


def kernel(concept_idx, concepts_weight):
    n, ncol = concept_idx.shape
    idx3 = concept_idx.astype(jnp.int32).reshape(
        NUM_WORKERS, n // NUM_WORKERS, ncol)
    return _lookup(concepts_weight.astype(jnp.float32), idx3)
